# baseline (speedup 1.0000x reference)
import functools

import jax
import jax.numpy as jnp
from jax import lax
from jax.experimental import pallas as pl
from jax.experimental.pallas import tpu as pltpu
from jax.experimental.pallas import tpu_sc as plsc

N = 10000
E = 320000
DIN = 128
DH = 32
DOUT = 32
TOPK = 16
NP0 = DIN * DH + DH
NP1 = DH * DOUT + DOUT

NC = 2
NS = 16
NW = NC * NS
NPAD = 10240
SLICE = NPAD // NS
E_PAD = 327680
E_PER_W = E_PAD // NW
CHUNK = 2048
N_CH = E_PER_W // CHUNK
SCHUNK = 1280
S_CH = E_PER_W // SCHUNK

_f32 = jnp.float32


def _mesh():
    return plsc.VectorSubcoreMesh(
        core_axis_name="c", subcore_axis_name="s", num_cores=NC, num_subcores=NS
    )


_SC_PARAMS = pltpu.CompilerParams(use_tc_tiling_on_sc=False)


def _deg_body(dst_hbm, zeros_hbm, ones_hbm, out_hbm, idx_v, ones_v, acc_sh, sem):
    c = lax.axis_index("c")
    s = lax.axis_index("s")
    wid = s * NC + c
    row0 = pl.multiple_of(s * SLICE, 8)
    pltpu.sync_copy(zeros_hbm, acc_sh.at[pl.ds(row0, SLICE)])
    pltpu.sync_copy(ones_hbm, ones_v)
    plsc.subcore_barrier()
    for k in range(N_CH):
        base = pl.multiple_of(wid * E_PER_W + k * CHUNK, 8)
        pltpu.sync_copy(dst_hbm.at[pl.ds(base, CHUNK)], idx_v)
        pltpu.sync_copy(ones_v, acc_sh.at[idx_v], add=True)
    plsc.subcore_barrier()
    pltpu.sync_copy(acc_sh.at[pl.ds(row0, SLICE)], out_hbm.at[c, pl.ds(row0, SLICE)])


def _degree(dst_p):
    zeros = jnp.zeros((SLICE,), _f32)
    ones = jnp.ones((CHUNK,), _f32)
    f = pl.kernel(
        _deg_body,
        out_type=jax.ShapeDtypeStruct((NC, NPAD), _f32),
        mesh=_mesh(),
        scratch_types=[
            pltpu.VMEM((CHUNK,), jnp.int32),
            pltpu.VMEM((CHUNK,), _f32),
            pltpu.VMEM_SHARED((NPAD,), _f32),
            pltpu.SemaphoreType.DMA,
        ],
        compiler_params=_SC_PARAMS,
    )
    return f(dst_p, zeros, ones)


def _edge_body(src_hbm, dst_hbm, xws_hbm, zeros_hbm, out_hbm,
               sidx0, sidx1, didx0, didx1, rows0, rows1, acc_sh, sem0, sem1):
    c = lax.axis_index("c")
    s = lax.axis_index("s")
    wid = s * NC + c
    row0 = pl.multiple_of(s * SLICE, 8)
    sidx = (sidx0, sidx1)
    didx = (didx0, didx1)
    rows = (rows0, rows1)
    sems = (sem0, sem1)
    pltpu.sync_copy(zeros_hbm, acc_sh.at[pl.ds(row0, SLICE)])

    def load(k, b):
        base = pl.multiple_of(wid * E_PER_W + k * SCHUNK, 8)
        pltpu.sync_copy(src_hbm.at[pl.ds(base, SCHUNK)], sidx[b])
        pltpu.sync_copy(dst_hbm.at[pl.ds(base, SCHUNK)], didx[b])
        return pltpu.async_copy(xws_hbm.at[sidx[b]], rows[b], sems[b])

    descs = [load(0, 0), None]
    plsc.subcore_barrier()
    for k in range(S_CH):
        b = k & 1
        if k + 1 < S_CH:
            descs[1 - b] = load(k + 1, 1 - b)
        descs[b].wait()
        pltpu.sync_copy(rows[b], acc_sh.at[didx[b]], add=True)
    plsc.subcore_barrier()
    pltpu.sync_copy(acc_sh.at[pl.ds(row0, SLICE)], out_hbm.at[c, pl.ds(row0, SLICE)])


def _edge_scatter(src_p, dst_p, xws):
    zeros = jnp.zeros((SLICE, DH), _f32)
    f = pl.kernel(
        _edge_body,
        out_type=jax.ShapeDtypeStruct((NC, NPAD, DH), _f32),
        mesh=_mesh(),
        scratch_types=[
            pltpu.VMEM((SCHUNK,), jnp.int32),
            pltpu.VMEM((SCHUNK,), jnp.int32),
            pltpu.VMEM((SCHUNK,), jnp.int32),
            pltpu.VMEM((SCHUNK,), jnp.int32),
            pltpu.VMEM((SCHUNK, DH), _f32),
            pltpu.VMEM((SCHUNK, DH), _f32),
            pltpu.VMEM_SHARED((NPAD, DH), _f32),
            pltpu.SemaphoreType.DMA,
            pltpu.SemaphoreType.DMA,
        ],
        compiler_params=_SC_PARAMS,
    )
    return f(src_p, dst_p, xws, zeros)


def _summ_body(h_ref, p_ref, z_ref, *, n_rows):
    p = p_ref[...]
    nrm = jnp.sqrt(jnp.sum(p * p))
    pn = p / (nrm + 1e-8)
    h = h_ref[...]
    y = lax.dot_general(pn, h, (((1,), (1,)), ((), ())),
                        preferred_element_type=_f32)
    iota = lax.broadcasted_iota(jnp.int32, (1, n_rows), 1)

    def body(k, y):
        m = jnp.max(y)
        idx = jnp.min(jnp.where(y == m, iota, n_rows))
        row = h_ref[pl.ds(idx, 1), :]
        z_ref[pl.ds(k, 1), :] = row * jnp.tanh(m)
        return jnp.where(iota == idx, -jnp.inf, y)

    lax.fori_loop(0, TOPK, body, y)


def _summarize(h, p):
    d = h.shape[1]
    return pl.pallas_call(
        functools.partial(_summ_body, n_rows=h.shape[0]),
        out_shape=jax.ShapeDtypeStruct((TOPK, d), _f32),
    )(h, p[None, :])


def _mv_body(wih_ref, whh_ref, z_ref, p_ref, gi_ref, gh_ref):
    gi_ref[...] = lax.dot_general(wih_ref[...], z_ref[...], (((1,), (0,)), ((), ())),
                                  preferred_element_type=_f32)
    gh_ref[...] = lax.dot_general(whh_ref[...], p_ref[...], (((1,), (0,)), ((), ())),
                                  preferred_element_type=_f32)


def _gru_matvec(Wih, Whh, z2, p2, blk):
    r3 = Wih.shape[0]
    gin = Wih.shape[1]
    np_ = Whh.shape[1]
    gi, gh = pl.pallas_call(
        _mv_body,
        grid=(r3 // blk,),
        in_specs=[
            pl.BlockSpec((blk, gin), lambda i: (i, 0)),
            pl.BlockSpec((blk, np_), lambda i: (i, 0)),
            pl.BlockSpec((gin, 1), lambda i: (0, 0)),
            pl.BlockSpec((np_, 1), lambda i: (0, 0)),
        ],
        out_specs=[
            pl.BlockSpec((blk, 1), lambda i: (i, 0)),
            pl.BlockSpec((blk, 1), lambda i: (i, 0)),
        ],
        out_shape=[
            jax.ShapeDtypeStruct((r3, 1), _f32),
            jax.ShapeDtypeStruct((r3, 1), _f32),
        ],
    )(Wih, Whh, z2, p2)
    return gi, gh


def _gate_body(gi_ref, gh_ref, prm_ref, bih_ref, bhh_ref, out_ref):
    gi = gi_ref[...] + bih_ref[...]
    gh = gh_ref[...] + bhh_ref[...]
    r = jax.nn.sigmoid(gi[0:1] + gh[0:1])
    zz = jax.nn.sigmoid(gi[1:2] + gh[1:2])
    n = jnp.tanh(gi[2:3] + r * gh[2:3])
    out_ref[...] = (1.0 - zz) * n + zz * prm_ref[...]


def _gru_gate(gi, gh, prm, bih, bhh):
    np_ = prm.shape[0]
    return pl.pallas_call(
        _gate_body,
        out_shape=jax.ShapeDtypeStruct((1, np_), _f32),
    )(gi.reshape(3, np_), gh.reshape(3, np_), prm[None, :],
      bih.reshape(3, np_), bhh.reshape(3, np_))


def _xw_body(h_ref, w_ref, dinv_ref, o_ref):
    o_ref[...] = lax.dot_general(h_ref[...], w_ref[...], (((1,), (0,)), ((), ())),
                                 preferred_element_type=_f32) * dinv_ref[...]


def _xw_scaled(h, W, dinv2):
    din = h.shape[1]
    rblk = 2000
    return pl.pallas_call(
        _xw_body,
        grid=(N // rblk,),
        in_specs=[
            pl.BlockSpec((rblk, din), lambda i: (i, 0)),
            pl.BlockSpec((din, DH), lambda i: (0, 0)),
            pl.BlockSpec((rblk, 1), lambda i: (i, 0)),
        ],
        out_specs=pl.BlockSpec((rblk, DH), lambda i: (i, 0)),
        out_shape=jax.ShapeDtypeStruct((NPAD, DH), _f32),
    )(h, W, dinv2)


def _combine_body(acc_ref, xws_ref, dinv_ref, b_ref, o_ref, *, relu):
    t = (acc_ref[0] + acc_ref[1] + xws_ref[...]) * dinv_ref[...] + b_ref[...]
    if relu:
        t = jnp.maximum(t, 0.0)
    o_ref[...] = t


def _combine(acc, xws, dinv2, b, relu):
    rblk = 2000
    return pl.pallas_call(
        functools.partial(_combine_body, relu=relu),
        grid=(N // rblk,),
        in_specs=[
            pl.BlockSpec((NC, rblk, DH), lambda i: (0, i, 0)),
            pl.BlockSpec((rblk, DH), lambda i: (i, 0)),
            pl.BlockSpec((rblk, 1), lambda i: (i, 0)),
            pl.BlockSpec((1, DH), lambda i: (0, 0)),
        ],
        out_specs=pl.BlockSpec((rblk, DH), lambda i: (i, 0)),
        out_shape=jax.ShapeDtypeStruct((N, DH), _f32),
    )(acc, xws, dinv2, b)


def _layer(h, p, W, b, Wih, Whh, bih, bhh, src_p, dst_p, dinv2, blk, relu):
    np_ = W.shape[0] * W.shape[1] + W.shape[1]
    Z = _summarize(h, p)
    z2 = Z.T.reshape(-1, 1)
    prm = jnp.concatenate([W.reshape(-1), b])
    gi, gh = _gru_matvec(Wih, Whh, z2, prm[:, None], blk)
    new = _gru_gate(gi, gh, prm, bih, bhh)[0]
    Wn = new[: W.shape[0] * W.shape[1]].reshape(W.shape)
    bn = new[W.shape[0] * W.shape[1]:]
    xws = _xw_scaled(h, Wn, dinv2)
    acc = _edge_scatter(src_p, dst_p, xws)
    return _combine(acc, xws, dinv2, bn[None, :], relu)


def kernel(x, edge_index, W0, b0, W1, b1, p0, p1,
           Wih0, Whh0, bih0, bhh0, Wih1, Whh1, bih1, bhh1):
    pad = jnp.full((E_PAD - E,), N, jnp.int32)
    src_p = jnp.concatenate([edge_index[0], pad])
    dst_p = jnp.concatenate([edge_index[1], pad])
    deg2 = _degree(dst_p)
    dinv = lax.rsqrt(deg2[0, :N] + deg2[1, :N] + 1.0)
    dinv2 = dinv[:, None]
    h1 = _layer(x, p0, W0, b0, Wih0, Whh0, bih0, bhh0,
                src_p, dst_p, dinv2, 1032, relu=True)
    out = _layer(h1, p1, W1, b1, Wih1, Whh1, bih1, bhh1,
                 src_p, dst_p, dinv2, 288, relu=False)
    return out


# speedup vs baseline: 1.0075x; 1.0075x over previous
import functools

import jax
import jax.numpy as jnp
from jax import lax
from jax.experimental import pallas as pl
from jax.experimental.pallas import tpu as pltpu
from jax.experimental.pallas import tpu_sc as plsc

N = 10000
E = 320000
DIN = 128
DH = 32
DOUT = 32
TOPK = 16
NP0 = DIN * DH + DH
NP1 = DH * DOUT + DOUT

NC = 2
NS = 16
NW = NC * NS
NPAD = 10240
SLICE = NPAD // NS
E_PAD = 327680
E_PER_W = E_PAD // NW
CHUNK = 2048
N_CH = E_PER_W // CHUNK
SCHUNK = 1280
S_CH = E_PER_W // SCHUNK

_f32 = jnp.float32


def _mesh():
    return plsc.VectorSubcoreMesh(
        core_axis_name="c", subcore_axis_name="s", num_cores=NC, num_subcores=NS
    )


_SC_PARAMS = pltpu.CompilerParams(use_tc_tiling_on_sc=False)


def _deg_body(dst_hbm, zeros_hbm, ones_hbm, out_hbm, idx_v, ones_v, acc_sh, sem):
    c = lax.axis_index("c")
    s = lax.axis_index("s")
    wid = s * NC + c
    row0 = pl.multiple_of(s * SLICE, 8)
    pltpu.sync_copy(zeros_hbm, acc_sh.at[pl.ds(row0, SLICE)])
    pltpu.sync_copy(ones_hbm, ones_v)
    plsc.subcore_barrier()
    for k in range(N_CH):
        base = pl.multiple_of(wid * E_PER_W + k * CHUNK, 8)
        pltpu.sync_copy(dst_hbm.at[pl.ds(base, CHUNK)], idx_v)
        pltpu.sync_copy(ones_v, acc_sh.at[idx_v], add=True)
    plsc.subcore_barrier()
    pltpu.sync_copy(acc_sh.at[pl.ds(row0, SLICE)], out_hbm.at[c, pl.ds(row0, SLICE)])


def _degree(dst_p):
    zeros = jnp.zeros((SLICE,), _f32)
    ones = jnp.ones((CHUNK,), _f32)
    f = pl.kernel(
        _deg_body,
        out_type=jax.ShapeDtypeStruct((NC, NPAD), _f32),
        mesh=_mesh(),
        scratch_types=[
            pltpu.VMEM((CHUNK,), jnp.int32),
            pltpu.VMEM((CHUNK,), _f32),
            pltpu.VMEM_SHARED((NPAD,), _f32),
            pltpu.SemaphoreType.DMA,
        ],
        compiler_params=_SC_PARAMS,
    )
    return f(dst_p, zeros, ones)


def _edge_body(src_hbm, dst_hbm, xws_hbm, zeros_hbm, out_hbm,
               sidx0, sidx1, didx0, didx1, rows0, rows1, acc_sh, sem0, sem1):
    c = lax.axis_index("c")
    s = lax.axis_index("s")
    wid = s * NC + c
    row0 = pl.multiple_of(s * SLICE, 8)
    sidx = (sidx0, sidx1)
    didx = (didx0, didx1)
    rows = (rows0, rows1)
    sems = (sem0, sem1)
    pltpu.sync_copy(zeros_hbm, acc_sh.at[pl.ds(row0, SLICE)])

    def load(k, b):
        base = pl.multiple_of(wid * E_PER_W + k * SCHUNK, 8)
        pltpu.sync_copy(src_hbm.at[pl.ds(base, SCHUNK)], sidx[b])
        pltpu.sync_copy(dst_hbm.at[pl.ds(base, SCHUNK)], didx[b])
        return pltpu.async_copy(xws_hbm.at[sidx[b]], rows[b], sems[b])

    descs = [load(0, 0), None]
    plsc.subcore_barrier()
    for k in range(S_CH):
        b = k & 1
        if k + 1 < S_CH:
            descs[1 - b] = load(k + 1, 1 - b)
        descs[b].wait()
        pltpu.sync_copy(rows[b], acc_sh.at[didx[b]], add=True)
    plsc.subcore_barrier()
    pltpu.sync_copy(acc_sh.at[pl.ds(row0, SLICE)], out_hbm.at[c, pl.ds(row0, SLICE)])


def _edge_scatter(src_p, dst_p, xws):
    zeros = jnp.zeros((SLICE, DH), _f32)
    f = pl.kernel(
        _edge_body,
        out_type=jax.ShapeDtypeStruct((NC, NPAD, DH), _f32),
        mesh=_mesh(),
        scratch_types=[
            pltpu.VMEM((SCHUNK,), jnp.int32),
            pltpu.VMEM((SCHUNK,), jnp.int32),
            pltpu.VMEM((SCHUNK,), jnp.int32),
            pltpu.VMEM((SCHUNK,), jnp.int32),
            pltpu.VMEM((SCHUNK, DH), _f32),
            pltpu.VMEM((SCHUNK, DH), _f32),
            pltpu.VMEM_SHARED((NPAD, DH), _f32),
            pltpu.SemaphoreType.DMA,
            pltpu.SemaphoreType.DMA,
        ],
        compiler_params=_SC_PARAMS,
    )
    return f(src_p, dst_p, xws, zeros)


def _summ_body(h_ref, p_ref, z_ref, *, n_rows):
    p = p_ref[...]
    nrm = jnp.sqrt(jnp.sum(p * p))
    pn = p / (nrm + 1e-8)
    h = h_ref[...]
    y = lax.dot_general(pn, h, (((1,), (1,)), ((), ())),
                        preferred_element_type=_f32)
    iota = lax.broadcasted_iota(jnp.int32, (1, n_rows), 1)

    def body(k, y):
        m = jnp.max(y)
        idx = jnp.min(jnp.where(y == m, iota, n_rows))
        row = h_ref[pl.ds(idx, 1), :]
        z_ref[pl.ds(k, 1), :] = row * jnp.tanh(m)
        return jnp.where(iota == idx, -jnp.inf, y)

    lax.fori_loop(0, TOPK, body, y)


def _summarize(h, p):
    d = h.shape[1]
    return pl.pallas_call(
        functools.partial(_summ_body, n_rows=h.shape[0]),
        out_shape=jax.ShapeDtypeStruct((TOPK, d), _f32),
    )(h, p[None, :])


def _mv_body(wih_ref, whh_ref, z_ref, p_ref, gi_ref, gh_ref):
    gi_ref[...] = lax.dot_general(wih_ref[...], z_ref[...], (((1,), (0,)), ((), ())),
                                  preferred_element_type=_f32)
    gh_ref[...] = lax.dot_general(whh_ref[...], p_ref[...], (((1,), (0,)), ((), ())),
                                  preferred_element_type=_f32)


def _gru_matvec(Wih, Whh, z2, p2, blk):
    r3 = Wih.shape[0]
    gin = Wih.shape[1]
    np_ = Whh.shape[1]
    gi, gh = pl.pallas_call(
        _mv_body,
        grid=(r3 // blk,),
        in_specs=[
            pl.BlockSpec((blk, gin), lambda i: (i, 0)),
            pl.BlockSpec((blk, np_), lambda i: (i, 0)),
            pl.BlockSpec((gin, 1), lambda i: (0, 0)),
            pl.BlockSpec((np_, 1), lambda i: (0, 0)),
        ],
        out_specs=[
            pl.BlockSpec((blk, 1), lambda i: (i, 0)),
            pl.BlockSpec((blk, 1), lambda i: (i, 0)),
        ],
        out_shape=[
            jax.ShapeDtypeStruct((r3, 1), _f32),
            jax.ShapeDtypeStruct((r3, 1), _f32),
        ],
    )(Wih, Whh, z2, p2)
    return gi, gh


def _gate_body(gi_ref, gh_ref, prm_ref, bih_ref, bhh_ref, out_ref):
    gi = gi_ref[...] + bih_ref[...]
    gh = gh_ref[...] + bhh_ref[...]
    r = jax.nn.sigmoid(gi[0:1] + gh[0:1])
    zz = jax.nn.sigmoid(gi[1:2] + gh[1:2])
    n = jnp.tanh(gi[2:3] + r * gh[2:3])
    out_ref[...] = (1.0 - zz) * n + zz * prm_ref[...]


def _gru_gate(gi, gh, prm, bih, bhh):
    np_ = prm.shape[0]
    return pl.pallas_call(
        _gate_body,
        out_shape=jax.ShapeDtypeStruct((1, np_), _f32),
    )(gi.reshape(3, np_), gh.reshape(3, np_), prm[None, :],
      bih.reshape(3, np_), bhh.reshape(3, np_))


def _xw_body(h_ref, w_ref, dinv_ref, o_ref):
    o_ref[...] = lax.dot_general(h_ref[...], w_ref[...], (((1,), (0,)), ((), ())),
                                 preferred_element_type=_f32) * dinv_ref[...]


def _xw_scaled(h, W, dinv2):
    din = h.shape[1]
    rblk = 2000
    return pl.pallas_call(
        _xw_body,
        grid=(N // rblk,),
        in_specs=[
            pl.BlockSpec((rblk, din), lambda i: (i, 0)),
            pl.BlockSpec((din, DH), lambda i: (0, 0)),
            pl.BlockSpec((rblk, 1), lambda i: (i, 0)),
        ],
        out_specs=pl.BlockSpec((rblk, DH), lambda i: (i, 0)),
        out_shape=jax.ShapeDtypeStruct((NPAD, DH), _f32),
    )(h, W, dinv2)


def _combine_body(acc_ref, xws_ref, dinv_ref, b_ref, o_ref, *, relu):
    t = (acc_ref[0] + acc_ref[1] + xws_ref[...]) * dinv_ref[...] + b_ref[...]
    if relu:
        t = jnp.maximum(t, 0.0)
    o_ref[...] = t


def _combine(acc, xws, dinv2, b, relu):
    rblk = 2000
    return pl.pallas_call(
        functools.partial(_combine_body, relu=relu),
        grid=(N // rblk,),
        in_specs=[
            pl.BlockSpec((NC, rblk, DH), lambda i: (0, i, 0)),
            pl.BlockSpec((rblk, DH), lambda i: (i, 0)),
            pl.BlockSpec((rblk, 1), lambda i: (i, 0)),
            pl.BlockSpec((1, DH), lambda i: (0, 0)),
        ],
        out_specs=pl.BlockSpec((rblk, DH), lambda i: (i, 0)),
        out_shape=jax.ShapeDtypeStruct((N, DH), _f32),
    )(acc, xws, dinv2, b)


def _layer(h, p, W, b, Wih, Whh, bih, bhh, src_p, dst_p, dinv2, blk, relu):
    np_ = W.shape[0] * W.shape[1] + W.shape[1]
    Z = _summarize(h, p)
    z2 = Z.T.reshape(-1, 1)
    prm = jnp.concatenate([W.reshape(-1), b])
    gi, gh = _gru_matvec(Wih, Whh, z2, prm[:, None], blk)
    new = _gru_gate(gi, gh, prm, bih, bhh)[0]
    Wn = new[: W.shape[0] * W.shape[1]].reshape(W.shape)
    bn = new[W.shape[0] * W.shape[1]:]
    xws = _xw_scaled(h, Wn, dinv2)
    acc = _edge_scatter(src_p, dst_p, xws)
    return _combine(acc, xws, dinv2, bn[None, :], relu)


def kernel(x, edge_index, W0, b0, W1, b1, p0, p1,
           Wih0, Whh0, bih0, bhh0, Wih1, Whh1, bih1, bhh1):
    pad = jnp.full((E_PAD - E,), N, jnp.int32)
    src_p = jnp.concatenate([edge_index[0], pad])
    dst_p = jnp.concatenate([edge_index[1], pad])
    deg2 = _degree(dst_p)
    dinv = lax.rsqrt(deg2[0, :N] + deg2[1, :N] + 1.0)
    dinv2 = dinv[:, None]
    h1 = _layer(x, p0, W0, b0, Wih0, Whh0, bih0, bhh0,
                src_p, dst_p, dinv2, 688, relu=True)
    out = _layer(h1, p1, W1, b1, Wih1, Whh1, bih1, bhh1,
                 src_p, dst_p, dinv2, 1056, relu=False)
    return out
